# baseline (speedup 1.0000x reference)
import functools

import jax
import jax.numpy as jnp
from jax import lax
from jax.experimental import pallas as pl
from jax.experimental.pallas import tpu as pltpu
from jax.experimental.pallas import tpu_sc as plsc

_N_EXP = 8
_D_IN = 1024
_D_HID = 512
_T = 2048
_B = 256
_NBR = 24
_CAP = _NBR * _B
_NBS = _T // _B
_NB = _NBR + _NBS
_NW = 32
_SLOTS_W = _CAP // _NW
_TOK_W = _T // _NW


def _top2(x, gate_w):
    logits = jax.lax.dot_general(
        x, gate_w, (((1,), (1,)), ((), ())), preferred_element_type=jnp.float32
    )
    m = jnp.max(logits, axis=-1, keepdims=True)
    p = jnp.exp(logits - m)
    s = p / jnp.sum(p, axis=-1, keepdims=True)
    lane = lax.broadcasted_iota(jnp.int32, s.shape, 1)
    m1 = jnp.max(s, axis=-1, keepdims=True)
    i1 = jnp.min(jnp.where(s >= m1, lane, _N_EXP), axis=-1, keepdims=True)
    s2 = jnp.where(lane == i1, -jnp.inf, s)
    m2 = jnp.max(s2, axis=-1, keepdims=True)
    i2 = jnp.min(jnp.where(s2 >= m2, lane, _N_EXP), axis=-1, keepdims=True)
    return m1, i1.astype(jnp.float32), m2, i2.astype(jnp.float32)


def _meta_body(x_ref, gate_ref, dest_ref, pw_ref, eid_ref, valid_ref):
    x = x_ref[...]
    m1, i1f, m2, i2f = _top2(x, gate_ref[...])
    pw_ref[:, 0:1] = m1
    pw_ref[:, 1:2] = m2

    lane8 = lax.broadcasted_iota(jnp.int32, (512, _N_EXP), 1).astype(jnp.float32)
    r_io = lax.broadcasted_iota(jnp.int32, (512, 512), 0)
    c_io = lax.broadcasted_iota(jnp.int32, (512, 512), 1)
    ltri = (r_io > c_io).astype(jnp.float32)

    base = jnp.zeros((1, _N_EXP), jnp.float32)
    ranks = []
    onehots = []
    for c in range(8):
        src = i1f if c < 4 else i2f
        r0 = (c % 4) * 512
        ev = src[r0:r0 + 512, :]
        oh = (ev == lane8).astype(jnp.float32)
        cum = lax.dot_general(
            ltri, oh, (((1,), (0,)), ((), ())), preferred_element_type=jnp.float32
        )
        ranks.append(jnp.sum(oh * (cum + base), axis=1, keepdims=True))
        onehots.append(oh)
        base = base + jnp.sum(oh, axis=0, keepdims=True)

    counts = base
    padded = jnp.ceil(counts * (1.0 / _B)) * _B
    u_r = lax.broadcasted_iota(jnp.int32, (_N_EXP, _N_EXP), 0)
    u_c = lax.broadcasted_iota(jnp.int32, (_N_EXP, _N_EXP), 1)
    utri = (u_r < u_c).astype(jnp.float32)
    offs = lax.dot_general(
        padded, utri, (((1,), (0,)), ((), ())), preferred_element_type=jnp.float32
    )

    for c in range(8):
        r0 = (c % 4) * 512
        col = 0 if c < 4 else 1
        dest_c = ranks[c] + jnp.sum(onehots[c] * offs, axis=1, keepdims=True)
        dest_ref[pl.ds(r0, 512), col:col + 1] = dest_c.astype(jnp.int32)

    ends = offs + padded
    b_row = lax.broadcasted_iota(jnp.int32, (_NB, 1), 0).astype(jnp.float32) * _B
    eid = jnp.sum((ends <= b_row).astype(jnp.float32), axis=1, keepdims=True)
    eid = jnp.minimum(eid, float(_N_EXP - 1))
    total = jnp.sum(padded, axis=1, keepdims=True)
    valid = (b_row < total).astype(jnp.int32)
    eid_ref[...] = eid.astype(jnp.int32)
    valid_ref[...] = valid


def _meta(xf, gate_w, interpret=False):
    return pl.pallas_call(
        _meta_body,
        in_specs=[
            pl.BlockSpec((_T, _D_IN), lambda: (0, 0)),
            pl.BlockSpec((_N_EXP, _D_IN), lambda: (0, 0)),
        ],
        out_specs=[
            pl.BlockSpec((_T, 2), lambda: (0, 0)),
            pl.BlockSpec((_T, 2), lambda: (0, 0)),
            pl.BlockSpec((_NB, 1), lambda: (0, 0)),
            pl.BlockSpec((_NB, 1), lambda: (0, 0)),
        ],
        out_shape=[
            jax.ShapeDtypeStruct((_T, 2), jnp.int32),
            jax.ShapeDtypeStruct((_T, 2), jnp.float32),
            jax.ShapeDtypeStruct((_NB, 1), jnp.int32),
            jax.ShapeDtypeStruct((_NB, 1), jnp.int32),
        ],
        interpret=interpret,
    )(xf, gate_w)


@functools.cache
def _dispatch_sc_kernel():
    mesh = plsc.VectorSubcoreMesh(core_axis_name="c", subcore_axis_name="s")
    return functools.partial(
        pl.kernel,
        out_type=(
            jax.ShapeDtypeStruct((_CAP, _D_IN), jnp.float32),
            jax.ShapeDtypeStruct((_CAP,), jnp.float32),
        ),
        mesh=mesh,
        scratch_types=[
            pltpu.VMEM((2 * _T,), jnp.int32),
            pltpu.VMEM((2 * _T,), jnp.float32),
            pltpu.VMEM((_CAP,), jnp.int32),
            pltpu.VMEM((_CAP,), jnp.float32),
            pltpu.VMEM_SHARED((_CAP,), jnp.int32),
            pltpu.VMEM((_SLOTS_W,), jnp.int32),
            pltpu.VMEM((_SLOTS_W // 2, _D_IN), jnp.float32),
            pltpu.SemaphoreType.DMA,
        ],
        compiler_params=pltpu.CompilerParams(needs_layout_passes=False),
    )(_dispatch_sc_body)


def _dispatch_sc_body(dest_hbm, pwp_hbm, x_hbm, xs_out, pw_out,
                      dest_v, pwp_v, tok_v, pws_v, tok_sh, idx_v, rows_v, sem):
    cid = lax.axis_index("c")
    sid = lax.axis_index("s")

    @pl.when(sid == 0)
    def _build():
        pltpu.sync_copy(dest_hbm, dest_v)
        pltpu.sync_copy(pwp_hbm, pwp_v)
        zero_i = jnp.zeros((16,), jnp.int32)
        zero_f = jnp.zeros((16,), jnp.float32)

        def zbody(i, carry):
            tok_v[pl.ds(i * 16, 16)] = zero_i
            pws_v[pl.ds(i * 16, 16)] = zero_f
            return carry

        lax.fori_loop(0, _CAP // 16, zbody, 0)

        def sbody(i, carry):
            d = dest_v[pl.ds(i * 16, 16)]
            j = lax.iota(jnp.int32, 16) + i * 16
            t = lax.shift_right_logical(j, 1)
            w = pwp_v[pl.ds(i * 16, 16)]
            plsc.store_scatter(tok_v, [d], t)
            plsc.store_scatter(pws_v, [d], w)
            return carry

        lax.fori_loop(0, 0, sbody, 0)
        pltpu.sync_copy(tok_v, tok_sh)

        @pl.when(cid == 0)
        def _():
            pltpu.sync_copy(pws_v, pw_out)

    plsc.subcore_barrier()

    wid = sid * 2 + cid
    base = wid * _SLOTS_W
    pltpu.sync_copy(tok_sh.at[pl.ds(base, _SLOTS_W)], idx_v)
    half = _SLOTS_W // 2
    for h in range(2):
        cp = pltpu.async_copy(x_hbm.at[idx_v.at[pl.ds(h * half, half)]], rows_v, sem)
        cp.wait()
        pltpu.sync_copy(rows_v, xs_out.at[pl.ds(base + h * half, half)])


def _swiglu(xs, w1, w2, scale):
    h = jax.lax.dot_general(
        xs, w1, (((1,), (1,)), ((), ())), preferred_element_type=jnp.float32
    )
    y = h[:, :_D_HID]
    g = h[:, _D_HID:]
    act = y * (g * jax.lax.logistic(g))
    if scale is not None:
        act = act * scale
    return jax.lax.dot_general(
        act, w2, (((1,), (1,)), ((), ())), preferred_element_type=jnp.float32
    )


def _gmm_body(eid_ref, valid_ref, xs_ref, xb_ref, pw_ref,
              w1_ref, w2_ref, sw1_ref, sw2_ref, y_ref):
    b = pl.program_id(0)

    @pl.when(b < _NBR)
    def _routed():
        @pl.when(valid_ref[b] != 0)
        def _():
            y_ref[...] = _swiglu(xs_ref[...], w1_ref[0], w2_ref[0], pw_ref[...])

    @pl.when(b >= _NBR)
    def _shared():
        y_ref[...] = _swiglu(xb_ref[...], sw1_ref[...], sw2_ref[...], None)


def _gmm(eid, valid, x_sorted, xf, pw2d,
         expert_fc1, expert_fc2, shared_fc1, shared_fc2, interpret=False):
    grid_spec = pltpu.PrefetchScalarGridSpec(
        num_scalar_prefetch=2,
        grid=(_NB,),
        in_specs=[
            pl.BlockSpec((_B, _D_IN), lambda b, e, v: (jnp.minimum(b, _NBR - 1), 0)),
            pl.BlockSpec((_B, _D_IN), lambda b, e, v: (jnp.clip(b - _NBR, 0, _NBS - 1), 0)),
            pl.BlockSpec((_B, 1), lambda b, e, v: (jnp.minimum(b, _NBR - 1), 0)),
            pl.BlockSpec((1, 2 * _D_HID, _D_IN), lambda b, e, v: (e[b], 0, 0)),
            pl.BlockSpec((1, _D_IN, _D_HID), lambda b, e, v: (e[b], 0, 0)),
            pl.BlockSpec((2 * _D_HID, _D_IN), lambda b, e, v: (0, 0)),
            pl.BlockSpec((_D_IN, _D_HID), lambda b, e, v: (0, 0)),
        ],
        out_specs=pl.BlockSpec((_B, _D_IN), lambda b, e, v: (b, 0)),
    )
    return pl.pallas_call(
        _gmm_body,
        grid_spec=grid_spec,
        out_shape=jax.ShapeDtypeStruct((_NB * _B, _D_IN), jnp.float32),
        compiler_params=pltpu.CompilerParams(
            dimension_semantics=("arbitrary",),
        ),
        interpret=interpret,
    )(eid, valid, x_sorted, xf, pw2d,
      expert_fc1, expert_fc2, shared_fc1, shared_fc2)


@functools.cache
def _combine_sc_kernel():
    mesh = plsc.VectorSubcoreMesh(core_axis_name="c", subcore_axis_name="s")
    return functools.partial(
        pl.kernel,
        out_type=jax.ShapeDtypeStruct((_T, _D_IN), jnp.float32),
        mesh=mesh,
        scratch_types=[
            pltpu.VMEM((_TOK_W // 2,), jnp.int32),
            pltpu.VMEM((_TOK_W // 2, _D_IN), jnp.float32),
            pltpu.VMEM((_TOK_W // 2, _D_IN), jnp.float32),
            pltpu.SemaphoreType.DMA,
        ],
        compiler_params=pltpu.CompilerParams(needs_layout_passes=False),
    )(_combine_sc_body)


def _combine_sc_body(d0_hbm, d1_hbm, y_hbm, z_out, d_v, acc_v, buf_v, sem):
    cid = lax.axis_index("c")
    sid = lax.axis_index("s")
    wid = sid * 2 + cid
    gt0 = wid * _TOK_W
    half = _TOK_W // 2

    def _accumulate(i, carry):
        for k in range(_D_IN // 16):
            sl = pl.ds(k * 16, 16)
            acc_v[i, sl] = acc_v[i, sl] + buf_v[i, sl]
        return carry

    for h in range(2):
        t0 = gt0 + h * half
        pltpu.sync_copy(d0_hbm.at[pl.ds(t0, half)], d_v)
        pltpu.async_copy(y_hbm.at[d_v], acc_v, sem).wait()
        pltpu.sync_copy(d1_hbm.at[pl.ds(t0, half)], d_v)
        pltpu.async_copy(y_hbm.at[d_v], buf_v, sem).wait()
        lax.fori_loop(0, half, _accumulate, 0)
        pltpu.sync_copy(y_hbm.at[pl.ds(_CAP + t0, half)], buf_v)
        lax.fori_loop(0, half, _accumulate, 0)
        pltpu.sync_copy(acc_v, z_out.at[pl.ds(t0, half)])


@jax.jit
def kernel(x, gate_w, expert_fc1, expert_fc2, shared_fc1, shared_fc2):
    xf = x.reshape(-1, _D_IN)
    dest2, pw2, eid, valid = _meta(xf, gate_w)
    dest_flat = dest2.reshape(-1)
    pw_pairs = pw2.reshape(-1)
    x_sorted, pw_slots = _dispatch_sc_kernel()(dest_flat, pw_pairs, xf)
    y = _gmm(eid.reshape(-1), valid.reshape(-1), x_sorted, xf,
             pw_slots.reshape(_CAP, 1),
             expert_fc1, expert_fc2, shared_fc1, shared_fc2)
    z = _combine_sc_kernel()(dest2[:, 0], dest2[:, 1], y)
    return z.reshape(x.shape)


# speedup vs baseline: 2.4762x; 2.4762x over previous
import functools

import jax
import jax.numpy as jnp
from jax import lax
from jax.experimental import pallas as pl
from jax.experimental.pallas import tpu as pltpu
from jax.experimental.pallas import tpu_sc as plsc

_N_EXP = 8
_D_IN = 1024
_D_HID = 512
_T = 2048
_B = 256
_NBR = 24
_CAP = _NBR * _B
_NBS = _T // _B
_NB = _NBR + _NBS
_NW = 32
_SLOTS_W = _CAP // _NW
_TOK_W = _T // _NW


def _top2(x, gate_w):
    logits = jax.lax.dot_general(
        x, gate_w, (((1,), (1,)), ((), ())), preferred_element_type=jnp.float32
    )
    m = jnp.max(logits, axis=-1, keepdims=True)
    p = jnp.exp(logits - m)
    s = p / jnp.sum(p, axis=-1, keepdims=True)
    lane = lax.broadcasted_iota(jnp.int32, s.shape, 1)
    m1 = jnp.max(s, axis=-1, keepdims=True)
    i1 = jnp.min(jnp.where(s >= m1, lane, _N_EXP), axis=-1, keepdims=True)
    s2 = jnp.where(lane == i1, -jnp.inf, s)
    m2 = jnp.max(s2, axis=-1, keepdims=True)
    i2 = jnp.min(jnp.where(s2 >= m2, lane, _N_EXP), axis=-1, keepdims=True)
    return m1, i1.astype(jnp.float32), m2, i2.astype(jnp.float32)


def _meta_body(x_ref, gate_ref, dest_ref, pw_ref, eid_ref, valid_ref):
    x = x_ref[...]
    m1, i1f, m2, i2f = _top2(x, gate_ref[...])
    pw_ref[:, 0:1] = m1
    pw_ref[:, 1:2] = m2

    lane8 = lax.broadcasted_iota(jnp.int32, (512, _N_EXP), 1).astype(jnp.float32)
    r_io = lax.broadcasted_iota(jnp.int32, (512, 512), 0)
    c_io = lax.broadcasted_iota(jnp.int32, (512, 512), 1)
    ltri = (r_io > c_io).astype(jnp.float32)

    base = jnp.zeros((1, _N_EXP), jnp.float32)
    ranks = []
    onehots = []
    for c in range(8):
        src = i1f if c < 4 else i2f
        r0 = (c % 4) * 512
        ev = src[r0:r0 + 512, :]
        oh = (ev == lane8).astype(jnp.float32)
        cum = lax.dot_general(
            ltri, oh, (((1,), (0,)), ((), ())), preferred_element_type=jnp.float32
        )
        ranks.append(jnp.sum(oh * (cum + base), axis=1, keepdims=True))
        onehots.append(oh)
        base = base + jnp.sum(oh, axis=0, keepdims=True)

    counts = base
    padded = jnp.ceil(counts * (1.0 / _B)) * _B
    u_r = lax.broadcasted_iota(jnp.int32, (_N_EXP, _N_EXP), 0)
    u_c = lax.broadcasted_iota(jnp.int32, (_N_EXP, _N_EXP), 1)
    utri = (u_r < u_c).astype(jnp.float32)
    offs = lax.dot_general(
        padded, utri, (((1,), (0,)), ((), ())), preferred_element_type=jnp.float32
    )

    for c in range(8):
        r0 = (c % 4) * 512
        col = 0 if c < 4 else 1
        dest_c = ranks[c] + jnp.sum(onehots[c] * offs, axis=1, keepdims=True)
        dest_ref[pl.ds(r0, 512), col:col + 1] = dest_c.astype(jnp.int32)

    ends = offs + padded
    b_row = lax.broadcasted_iota(jnp.int32, (_NB, 1), 0).astype(jnp.float32) * _B
    eid = jnp.sum((ends <= b_row).astype(jnp.float32), axis=1, keepdims=True)
    eid = jnp.minimum(eid, float(_N_EXP - 1))
    total = jnp.sum(padded, axis=1, keepdims=True)
    valid = (b_row < total).astype(jnp.int32)
    eid_ref[...] = eid.astype(jnp.int32)
    valid_ref[...] = valid


def _meta(xf, gate_w, interpret=False):
    return pl.pallas_call(
        _meta_body,
        in_specs=[
            pl.BlockSpec((_T, _D_IN), lambda: (0, 0)),
            pl.BlockSpec((_N_EXP, _D_IN), lambda: (0, 0)),
        ],
        out_specs=[
            pl.BlockSpec((_T, 2), lambda: (0, 0)),
            pl.BlockSpec((_T, 2), lambda: (0, 0)),
            pl.BlockSpec((_NB, 1), lambda: (0, 0)),
            pl.BlockSpec((_NB, 1), lambda: (0, 0)),
        ],
        out_shape=[
            jax.ShapeDtypeStruct((_T, 2), jnp.int32),
            jax.ShapeDtypeStruct((_T, 2), jnp.float32),
            jax.ShapeDtypeStruct((_NB, 1), jnp.int32),
            jax.ShapeDtypeStruct((_NB, 1), jnp.int32),
        ],
        interpret=interpret,
    )(xf, gate_w)


@functools.cache
def _dispatch_sc_kernel():
    mesh = plsc.VectorSubcoreMesh(core_axis_name="c", subcore_axis_name="s")
    return functools.partial(
        pl.kernel,
        out_type=(
            jax.ShapeDtypeStruct((_CAP, _D_IN), jnp.float32),
            jax.ShapeDtypeStruct((_CAP,), jnp.float32),
        ),
        mesh=mesh,
        scratch_types=[
            pltpu.VMEM((2 * _T,), jnp.int32),
            pltpu.VMEM((2 * _T,), jnp.float32),
            pltpu.VMEM((_CAP,), jnp.int32),
            pltpu.VMEM((_CAP,), jnp.float32),
            pltpu.VMEM_SHARED((_CAP,), jnp.int32),
            pltpu.VMEM((_SLOTS_W,), jnp.int32),
            pltpu.VMEM((_SLOTS_W // 2, _D_IN), jnp.float32),
            pltpu.SemaphoreType.DMA,
        ],
        compiler_params=pltpu.CompilerParams(needs_layout_passes=False),
    )(_dispatch_sc_body)


def _dispatch_sc_body(dest_hbm, pwp_hbm, x_hbm, xs_out, pw_out,
                      dest_v, pwp_v, tok_v, pws_v, tok_sh, idx_v, rows_v, sem):
    cid = lax.axis_index("c")
    sid = lax.axis_index("s")

    @pl.when(sid == 0)
    def _build():
        pltpu.sync_copy(dest_hbm, dest_v)
        pltpu.sync_copy(pwp_hbm, pwp_v)
        zero_f = jnp.zeros((16,), jnp.float32)

        def zbody(i, carry):
            tok_v[pl.ds(i * 16, 16)] = jnp.bitwise_and(
                lax.iota(jnp.int32, 16) + i * 16, _T - 1
            )
            pws_v[pl.ds(i * 16, 16)] = zero_f
            return carry

        lax.fori_loop(0, _CAP // 16, zbody, 0)

        def sbody(i, carry):
            d = dest_v[pl.ds(i * 16, 16)]
            j = lax.iota(jnp.int32, 16) + i * 16
            t = lax.shift_right_logical(j, 1)
            w = pwp_v[pl.ds(i * 16, 16)]
            plsc.store_scatter(tok_v, [d], t)
            plsc.store_scatter(pws_v, [d], w)
            return carry

        lax.fori_loop(0, (2 * _T) // 16, sbody, 0)
        pltpu.sync_copy(tok_v, tok_sh)

        @pl.when(cid == 0)
        def _():
            pltpu.sync_copy(pws_v, pw_out)

    plsc.subcore_barrier()

    wid = sid * 2 + cid
    base = wid * _SLOTS_W
    pltpu.sync_copy(tok_sh.at[pl.ds(base, _SLOTS_W)], idx_v)
    half = _SLOTS_W // 2
    for h in range(2):
        cp = pltpu.async_copy(x_hbm.at[idx_v.at[pl.ds(h * half, half)]], rows_v, sem)
        cp.wait()
        pltpu.sync_copy(rows_v, xs_out.at[pl.ds(base + h * half, half)])


def _swiglu(xs, w1, w2, scale):
    h = jax.lax.dot_general(
        xs, w1, (((1,), (1,)), ((), ())), preferred_element_type=jnp.float32
    )
    y = h[:, :_D_HID]
    g = h[:, _D_HID:]
    act = y * (g * jax.lax.logistic(g))
    if scale is not None:
        act = act * scale
    return jax.lax.dot_general(
        act, w2, (((1,), (1,)), ((), ())), preferred_element_type=jnp.float32
    )


def _gmm_body(eid_ref, valid_ref, xs_ref, xb_ref, pw_ref,
              w1_ref, w2_ref, sw1_ref, sw2_ref, y_ref):
    b = pl.program_id(0)

    @pl.when(b < _NBR)
    def _routed():
        @pl.when(valid_ref[b] != 0)
        def _():
            y_ref[...] = _swiglu(xs_ref[...], w1_ref[0], w2_ref[0], pw_ref[...])

    @pl.when(b >= _NBR)
    def _shared():
        y_ref[...] = _swiglu(xb_ref[...], sw1_ref[...], sw2_ref[...], None)


def _gmm(eid, valid, x_sorted, xf, pw2d,
         expert_fc1, expert_fc2, shared_fc1, shared_fc2, interpret=False):
    grid_spec = pltpu.PrefetchScalarGridSpec(
        num_scalar_prefetch=2,
        grid=(_NB,),
        in_specs=[
            pl.BlockSpec((_B, _D_IN), lambda b, e, v: (jnp.minimum(b, _NBR - 1), 0)),
            pl.BlockSpec((_B, _D_IN), lambda b, e, v: (jnp.clip(b - _NBR, 0, _NBS - 1), 0)),
            pl.BlockSpec((_B, 1), lambda b, e, v: (jnp.minimum(b, _NBR - 1), 0)),
            pl.BlockSpec((1, 2 * _D_HID, _D_IN), lambda b, e, v: (e[b], 0, 0)),
            pl.BlockSpec((1, _D_IN, _D_HID), lambda b, e, v: (e[b], 0, 0)),
            pl.BlockSpec((2 * _D_HID, _D_IN), lambda b, e, v: (0, 0)),
            pl.BlockSpec((_D_IN, _D_HID), lambda b, e, v: (0, 0)),
        ],
        out_specs=pl.BlockSpec((_B, _D_IN), lambda b, e, v: (b, 0)),
    )
    return pl.pallas_call(
        _gmm_body,
        grid_spec=grid_spec,
        out_shape=jax.ShapeDtypeStruct((_NB * _B, _D_IN), jnp.float32),
        compiler_params=pltpu.CompilerParams(
            dimension_semantics=("arbitrary",),
        ),
        interpret=interpret,
    )(eid, valid, x_sorted, xf, pw2d,
      expert_fc1, expert_fc2, shared_fc1, shared_fc2)


@functools.cache
def _combine_sc_kernel():
    mesh = plsc.VectorSubcoreMesh(core_axis_name="c", subcore_axis_name="s")
    return functools.partial(
        pl.kernel,
        out_type=jax.ShapeDtypeStruct((_T, _D_IN), jnp.float32),
        mesh=mesh,
        scratch_types=[
            pltpu.VMEM((_TOK_W // 2,), jnp.int32),
            pltpu.VMEM((_TOK_W // 2, _D_IN), jnp.float32),
            pltpu.VMEM((_TOK_W // 2, _D_IN), jnp.float32),
            pltpu.SemaphoreType.DMA,
        ],
        compiler_params=pltpu.CompilerParams(needs_layout_passes=False),
    )(_combine_sc_body)


def _combine_sc_body(d0_hbm, d1_hbm, y_hbm, z_out, d_v, acc_v, buf_v, sem):
    cid = lax.axis_index("c")
    sid = lax.axis_index("s")
    wid = sid * 2 + cid
    gt0 = wid * _TOK_W
    half = _TOK_W // 2

    def _accumulate(i, carry):
        for k in range(_D_IN // 16):
            sl = pl.ds(k * 16, 16)
            acc_v[i, sl] = acc_v[i, sl] + buf_v[i, sl]
        return carry

    for h in range(2):
        t0 = gt0 + h * half
        pltpu.sync_copy(d0_hbm.at[pl.ds(t0, half)], d_v)
        pltpu.async_copy(y_hbm.at[d_v], acc_v, sem).wait()
        pltpu.sync_copy(d1_hbm.at[pl.ds(t0, half)], d_v)
        pltpu.async_copy(y_hbm.at[d_v], buf_v, sem).wait()
        lax.fori_loop(0, half, _accumulate, 0)
        pltpu.sync_copy(y_hbm.at[pl.ds(_CAP + t0, half)], buf_v)
        lax.fori_loop(0, half, _accumulate, 0)
        pltpu.sync_copy(acc_v, z_out.at[pl.ds(t0, half)])


@jax.jit
def kernel(x, gate_w, expert_fc1, expert_fc2, shared_fc1, shared_fc2):
    xf = x.reshape(-1, _D_IN)
    dest2, pw2, eid, valid = _meta(xf, gate_w)
    dest_flat = dest2.reshape(-1)
    pw_pairs = pw2.reshape(-1)
    x_sorted, pw_slots = _dispatch_sc_kernel()(dest_flat, pw_pairs, xf)
    y = _gmm(eid.reshape(-1), valid.reshape(-1), x_sorted, xf,
             pw_slots.reshape(_CAP, 1),
             expert_fc1, expert_fc2, shared_fc1, shared_fc2)
    z = _combine_sc_kernel()(dest2[:, 0], dest2[:, 1], y)
    return z.reshape(x.shape)


# speedup vs baseline: 4.2677x; 1.7235x over previous
import jax
import jax.numpy as jnp
from jax.experimental import pallas as pl
from jax.experimental.pallas import tpu as pltpu

_N_EXP = 8
_D_IN = 1024
_D_HID = 512


def _top2(x, gate_w):
    logits = jax.lax.dot_general(
        x, gate_w, (((1,), (1,)), ((), ())), preferred_element_type=jnp.float32
    )
    m = jnp.max(logits, axis=-1, keepdims=True)
    p = jnp.exp(logits - m)
    s = p / jnp.sum(p, axis=-1, keepdims=True)
    lane = jax.lax.broadcasted_iota(jnp.int32, s.shape, 1)
    m1 = jnp.max(s, axis=-1, keepdims=True)
    i1 = jnp.min(jnp.where(s >= m1, lane, _N_EXP), axis=-1, keepdims=True)
    s2 = jnp.where(lane == i1, -jnp.inf, s)
    m2 = jnp.max(s2, axis=-1, keepdims=True)
    i2 = jnp.min(jnp.where(s2 >= m2, lane, _N_EXP), axis=-1, keepdims=True)
    return m1, i1.astype(jnp.float32), m2, i2.astype(jnp.float32)


def _swiglu(xb, w1, w2, scale):
    h = jax.lax.dot_general(
        xb, w1, (((1,), (1,)), ((), ())), preferred_element_type=jnp.float32
    )
    y = h[:, :_D_HID]
    g = h[:, _D_HID:]
    act = y * (g * jax.lax.logistic(g))
    if scale is not None:
        act = act * scale
    return jax.lax.dot_general(
        act, w2, (((1,), (1,)), ((), ())),
        preferred_element_type=jnp.float32,
    )


def _moe_body(x_ref, gate_ref, w1_ref, w2_ref, sw1_ref, sw2_ref, z_ref,
              m1_s, i1_s, m2_s, i2_s):
    e = pl.program_id(0)
    x = x_ref[...]

    @pl.when(e == 0)
    def _gate():
        m1, i1, m2, i2 = _top2(x, gate_ref[...])
        m1_s[...] = m1
        i1_s[...] = i1
        m2_s[...] = m2
        i2_s[...] = i2

    @pl.when(e < _N_EXP)
    def _routed():
        ef = e.astype(jnp.float32)
        w_e = (
            jnp.where(i1_s[...] == ef, m1_s[...], 0.0)
            + jnp.where(i2_s[...] == ef, m2_s[...], 0.0)
        )
        contrib = _swiglu(x, w1_ref[0], w2_ref[0], w_e)

        @pl.when(e == 0)
        def _():
            z_ref[...] = contrib

        @pl.when(e > 0)
        def _():
            z_ref[...] += contrib

    @pl.when(e == _N_EXP)
    def _shared():
        z_ref[...] += _swiglu(x, sw1_ref[...], sw2_ref[...], None)


def _moe(xf, gate_w, expert_fc1, expert_fc2, shared_fc1, shared_fc2, interpret=False):
    t = xf.shape[0]
    last = _N_EXP - 1
    return pl.pallas_call(
        _moe_body,
        grid=(9,),
        in_specs=[
            pl.BlockSpec((t, _D_IN), lambda e: (0, 0)),
            pl.BlockSpec((_N_EXP, _D_IN), lambda e: (0, 0)),
            pl.BlockSpec((1, 2 * _D_HID, _D_IN), lambda e: (jnp.minimum(e, last), 0, 0)),
            pl.BlockSpec((1, _D_IN, _D_HID), lambda e: (jnp.minimum(e, last), 0, 0)),
            pl.BlockSpec((2 * _D_HID, _D_IN), lambda e: (0, 0)),
            pl.BlockSpec((_D_IN, _D_HID), lambda e: (0, 0)),
        ],
        out_specs=pl.BlockSpec((t, _D_IN), lambda e: (0, 0)),
        out_shape=jax.ShapeDtypeStruct((t, _D_IN), jnp.float32),
        scratch_shapes=[pltpu.VMEM((t, 1), jnp.float32) for _ in range(4)],
        compiler_params=pltpu.CompilerParams(
            dimension_semantics=("arbitrary",),
        ),
        interpret=interpret,
    )(xf, gate_w, expert_fc1, expert_fc2, shared_fc1, shared_fc2)


@jax.jit
def kernel(x, gate_w, expert_fc1, expert_fc2, shared_fc1, shared_fc2):
    xf = x.reshape(-1, _D_IN)
    z = _moe(xf, gate_w, expert_fc1, expert_fc2, shared_fc1, shared_fc2)
    return z.reshape(x.shape)
